# baseline (speedup 1.0000x reference)
import functools

import jax
import jax.numpy as jnp
from jax import lax
from jax.experimental import pallas as pl
from jax.experimental.pallas import tpu as pltpu
from jax.experimental.pallas import tpu_sc as plsc

NUM_CORES = 2
NUM_SUBCORES = 16
LANES = 16
NW = NUM_CORES * NUM_SUBCORES

BATCH = 16384
CHUNK = 128
BPW = BATCH // NW
ROWS_PW = BPW // CHUNK
ROWW = 16
NUSERS = 1000000
NMOVIES = 100000
UPAD = (-NUSERS) % ROWW + ((-(NUSERS // ROWW + (NUSERS % ROWW > 0))) % 8) * ROWW
MPAD = (-NMOVIES) % ROWW + ((-(NMOVIES // ROWW + (NMOVIES % ROWW > 0))) % 8) * ROWW


@functools.partial(
    pl.kernel,
    mesh=plsc.VectorSubcoreMesh(core_axis_name="c", subcore_axis_name="s"),
    out_type=jax.ShapeDtypeStruct((BATCH,), jnp.float32),
    compiler_params=pltpu.CompilerParams(use_tc_tiling_on_sc=False,
                                         needs_layout_passes=False),
    scratch_types=[
        pltpu.VMEM((BPW,), jnp.int32),
        pltpu.VMEM((BPW,), jnp.int32),
        pltpu.VMEM((BPW,), jnp.int32),
        pltpu.VMEM((BPW,), jnp.int32),
        pltpu.VMEM((BPW, ROWW), jnp.float32),
        pltpu.VMEM((BPW, ROWW), jnp.float32),
        pltpu.VMEM((BPW,), jnp.float32),
        pltpu.VMEM((LANES,), jnp.float32),
        pltpu.SemaphoreType.DMA,
    ],
)
def _nbm_kernel(user_hbm, movie_hbm, ubias_hbm, mbias_hbm, gb_hbm, out_hbm,
                uidx, midx, urow, mrow, urv, mrv, outv, gbv, sem):
    wid = lax.axis_index("s") * NUM_CORES + lax.axis_index("c")
    base = wid * BPW
    pltpu.sync_copy(user_hbm.at[pl.ds(base, BPW)], uidx)
    pltpu.sync_copy(movie_hbm.at[pl.ds(base, BPW)], midx)
    pltpu.sync_copy(gb_hbm, gbv)
    for i in range(BPW // LANES):
        sl = pl.ds(i * LANES, LANES)
        urow[sl] = lax.shift_right_logical(uidx[sl], 4)
        mrow[sl] = lax.shift_right_logical(midx[sl], 4)
    copies = []
    for j in range(ROWS_PW):
        isl = pl.ds(j * CHUNK, CHUNK)
        copies.append(pltpu.async_copy(ubias_hbm.at[urow.at[isl]], urv.at[isl], sem))
        copies.append(pltpu.async_copy(mbias_hbm.at[mrow.at[isl]], mrv.at[isl], sem))
    for c in copies:
        c.wait()
    g = gbv[...]
    iota = lax.iota(jnp.int32, LANES)
    fifteen = jnp.full((LANES,), 15, jnp.int32)
    for j in range(ROWS_PW):
        for i in range(CHUNK // LANES):
            k = j * CHUNK + i * LANES
            sl = pl.ds(k, LANES)
            rloc = iota + k
            uv = plsc.load_gather(urv, [rloc, lax.bitwise_and(uidx[sl], fifteen)])
            mv = plsc.load_gather(mrv, [rloc, lax.bitwise_and(midx[sl], fifteen)])
            outv[sl] = uv + mv + g
    pltpu.sync_copy(outv, out_hbm.at[pl.ds(base, BPW)])


def kernel(user, movie, user_biases, movie_biases, global_bias):
    ub = jnp.pad(user_biases, ((0, UPAD), (0, 0))).reshape(-1, ROWW)
    mb = jnp.pad(movie_biases, ((0, MPAD), (0, 0))).reshape(-1, ROWW)
    gb = jnp.broadcast_to(global_bias.reshape(1), (LANES,))
    return _nbm_kernel(user, movie, ub, mb, gb)


# speedup vs baseline: 1.0029x; 1.0029x over previous
import functools

import jax
import jax.numpy as jnp
from jax import lax
from jax.experimental import pallas as pl
from jax.experimental.pallas import tpu as pltpu
from jax.experimental.pallas import tpu_sc as plsc

NUM_CORES = 2
NUM_SUBCORES = 16
LANES = 16
NW = NUM_CORES * NUM_SUBCORES

BATCH = 16384
CHUNK = 128
BPW = BATCH // NW
ROWS_PW = BPW // CHUNK
ROWW = 16
NUSERS = 1000000
NMOVIES = 100000
UPAD = (-NUSERS) % ROWW + ((-(NUSERS // ROWW + (NUSERS % ROWW > 0))) % 8) * ROWW
MPAD = (-NMOVIES) % ROWW + ((-(NMOVIES // ROWW + (NMOVIES % ROWW > 0))) % 8) * ROWW


@functools.partial(
    pl.kernel,
    mesh=plsc.VectorSubcoreMesh(core_axis_name="c", subcore_axis_name="s"),
    out_type=jax.ShapeDtypeStruct((BATCH,), jnp.float32),
    compiler_params=pltpu.CompilerParams(use_tc_tiling_on_sc=False,
                                         needs_layout_passes=False),
    scratch_types=[
        pltpu.VMEM((BPW,), jnp.int32),
        pltpu.VMEM((BPW,), jnp.int32),
        pltpu.VMEM((BPW,), jnp.int32),
        pltpu.VMEM((BPW,), jnp.int32),
        pltpu.VMEM((BPW, ROWW), jnp.float32),
        pltpu.VMEM((BPW, ROWW), jnp.float32),
        pltpu.VMEM((BPW,), jnp.float32),
        pltpu.VMEM((LANES,), jnp.float32),
        pltpu.SemaphoreType.DMA,
    ],
)
def _nbm_kernel(user_hbm, movie_hbm, ubias_hbm, mbias_hbm, gb_hbm, out_hbm,
                uidx, midx, urow, mrow, urv, mrv, outv, gbv, sem):
    wid = lax.axis_index("s") * NUM_CORES + lax.axis_index("c")
    base = wid * BPW
    pltpu.sync_copy(user_hbm.at[pl.ds(base, BPW)], uidx)
    pltpu.sync_copy(movie_hbm.at[pl.ds(base, BPW)], midx)
    pltpu.sync_copy(gb_hbm, gbv)
    for i in range(BPW // LANES):
        sl = pl.ds(i * LANES, LANES)
        urow[sl] = lax.shift_right_logical(uidx[sl], 4)
        mrow[sl] = lax.shift_right_logical(midx[sl], 4)
    copies = []
    for j in range(ROWS_PW):
        isl = pl.ds(j * CHUNK, CHUNK)
        copies.append(pltpu.async_copy(ubias_hbm.at[urow.at[isl]], urv.at[isl], sem))
        copies.append(pltpu.async_copy(mbias_hbm.at[mrow.at[isl]], mrv.at[isl], sem))
    for c in copies:
        c.wait()
    g = gbv[...]
    iota = lax.iota(jnp.int32, LANES)
    fifteen = jnp.full((LANES,), 15, jnp.int32)
    for j in range(ROWS_PW):
        for i in range(CHUNK // LANES):
            k = j * CHUNK + i * LANES
            sl = pl.ds(k, LANES)
            rloc = iota + k
            uv = plsc.load_gather(urv, [rloc, lax.bitwise_and(uidx[sl], fifteen)])
            mv = plsc.load_gather(mrv, [rloc, lax.bitwise_and(midx[sl], fifteen)])
            outv[sl] = uv + mv + g
    pltpu.sync_copy(outv, out_hbm.at[pl.ds(base, BPW)])


def kernel(user, movie, user_biases, movie_biases, global_bias):
    ub = user_biases.reshape(-1, ROWW)
    mb = movie_biases.reshape(-1, ROWW)
    gb = jnp.broadcast_to(global_bias.reshape(1), (LANES,))
    return _nbm_kernel(user, movie, ub, mb, gb)


# speedup vs baseline: 1.0909x; 1.0877x over previous
import functools

import jax
import jax.numpy as jnp
from jax import lax
from jax.experimental import pallas as pl
from jax.experimental.pallas import tpu as pltpu
from jax.experimental.pallas import tpu_sc as plsc

NUM_CORES = 2
NUM_SUBCORES = 16
LANES = 16
NW = NUM_CORES * NUM_SUBCORES

BATCH = 16384
CHUNK = 128
BPW = BATCH // NW
ROWS_PW = BPW // CHUNK


@functools.partial(
    pl.kernel,
    mesh=plsc.VectorSubcoreMesh(core_axis_name="c", subcore_axis_name="s"),
    out_type=jax.ShapeDtypeStruct((BATCH,), jnp.float32),
    scratch_types=[
        pltpu.VMEM((BPW,), jnp.int32),
        pltpu.VMEM((BPW,), jnp.int32),
        pltpu.VMEM((ROWS_PW, CHUNK), jnp.float32),
        pltpu.VMEM((ROWS_PW, CHUNK), jnp.float32),
        pltpu.VMEM((BPW,), jnp.float32),
        pltpu.VMEM((LANES,), jnp.float32),
        pltpu.SemaphoreType.DMA,
    ],
    compiler_params=pltpu.CompilerParams(use_tc_tiling_on_sc=False,
                                         needs_layout_passes=False),
)
def _nbm_kernel(user_hbm, movie_hbm, ubias_hbm, mbias_hbm, gb_hbm, out_hbm,
                uidx, midx, uval, mval, outv, gbv, sem):
    ub1 = ubias_hbm.at[0]
    mb1 = mbias_hbm.at[0]
    wid = lax.axis_index("s") * NUM_CORES + lax.axis_index("c")
    base = wid * BPW
    pltpu.sync_copy(user_hbm.at[pl.ds(base, BPW)], uidx)
    pltpu.sync_copy(movie_hbm.at[pl.ds(base, BPW)], midx)
    pltpu.sync_copy(gb_hbm, gbv)
    copies = []
    for j in range(ROWS_PW):
        isl = pl.ds(j * CHUNK, CHUNK)
        copies.append(pltpu.async_copy(ub1.at[uidx.at[isl]], uval.at[j], sem))
        copies.append(pltpu.async_copy(mb1.at[midx.at[isl]], mval.at[j], sem))
    for c in copies:
        c.wait()
    g = gbv[...]
    for j in range(ROWS_PW):
        for i in range(CHUNK // LANES):
            sl = pl.ds(i * LANES, LANES)
            outv[pl.ds(j * CHUNK + i * LANES, LANES)] = (
                uval[j, sl] + mval[j, sl] + g)
    pltpu.sync_copy(outv, out_hbm.at[pl.ds(base, BPW)])


def kernel(user, movie, user_biases, movie_biases, global_bias):
    ub = user_biases.reshape(1, -1)
    mb = movie_biases.reshape(1, -1)
    gb = jnp.broadcast_to(global_bias.reshape(1), (LANES,))
    return _nbm_kernel(user, movie, ub, mb, gb)


# speedup vs baseline: 1.1056x; 1.0134x over previous
import functools

import jax
import jax.numpy as jnp
from jax import lax
from jax.experimental import pallas as pl
from jax.experimental.pallas import tpu as pltpu
from jax.experimental.pallas import tpu_sc as plsc

NUM_CORES = 2
NUM_SUBCORES = 16
LANES = 16
NW = NUM_CORES * NUM_SUBCORES

BATCH = 16384
CHUNK = 128
BPW = BATCH // NW
ROWS_PW = BPW // CHUNK


@functools.partial(
    pl.kernel,
    mesh=plsc.VectorSubcoreMesh(core_axis_name="c", subcore_axis_name="s"),
    out_type=jax.ShapeDtypeStruct((BATCH,), jnp.float32),
    scratch_types=[
        pltpu.VMEM((BPW,), jnp.int32),
        pltpu.VMEM((BPW,), jnp.int32),
        pltpu.VMEM((ROWS_PW, CHUNK), jnp.float32),
        pltpu.VMEM((ROWS_PW, CHUNK), jnp.float32),
        pltpu.VMEM((BPW,), jnp.float32),
        pltpu.VMEM((LANES,), jnp.float32),
        pltpu.SemaphoreType.DMA,
    ],
    compiler_params=pltpu.CompilerParams(use_tc_tiling_on_sc=False,
                                         needs_layout_passes=False),
)
def _nbm_kernel(user_hbm, movie_hbm, ubias_hbm, mbias_hbm, gb_hbm, out_hbm,
                uidx, midx, uval, mval, outv, gbv, sem):
    ub1 = ubias_hbm.at[0]
    mb1 = mbias_hbm.at[0]
    wid = lax.axis_index("s") * NUM_CORES + lax.axis_index("c")
    base = wid * BPW
    ci1 = pltpu.async_copy(user_hbm.at[pl.ds(base, BPW)], uidx, sem)
    ci2 = pltpu.async_copy(movie_hbm.at[pl.ds(base, BPW)], midx, sem)
    pltpu.sync_copy(gb_hbm, gbv)
    ci1.wait()
    ci2.wait()
    copies = []
    for j in range(ROWS_PW):
        isl = pl.ds(j * CHUNK, CHUNK)
        copies.append(pltpu.async_copy(ub1.at[uidx.at[isl]], uval.at[j], sem))
        copies.append(pltpu.async_copy(mb1.at[midx.at[isl]], mval.at[j], sem))
    for c in copies:
        c.wait()
    g = gbv[...]
    for j in range(ROWS_PW):
        for i in range(CHUNK // LANES):
            sl = pl.ds(i * LANES, LANES)
            outv[pl.ds(j * CHUNK + i * LANES, LANES)] = (
                uval[j, sl] + mval[j, sl] + g)
    pltpu.sync_copy(outv, out_hbm.at[pl.ds(base, BPW)])


def kernel(user, movie, user_biases, movie_biases, global_bias):
    ub = user_biases.reshape(1, -1)
    mb = movie_biases.reshape(1, -1)
    gb = jnp.broadcast_to(global_bias.reshape(1), (LANES,))
    return _nbm_kernel(user, movie, ub, mb, gb)
